# baseline (speedup 1.0000x reference)
import functools

import jax
import jax.numpy as jnp
from jax import lax
from jax.experimental import pallas as pl
from jax.experimental.pallas import tpu as pltpu
from jax.experimental.pallas import tpu_sc as plsc

_B = 4
_N = 2048
_F = 1024
_P = 128

_NC = 2
_NS = 16
_NW = _NC * _NS
_SCB = 1
_CP = _SCB * _P // _NW
_VR = _N // 16


def _proj_kernel(y_pred_ref, y_real_ref, proj_ref, zp_ref, zr_ref):
    proj = proj_ref[...]
    dn = (((0,), (1,)), ((), ()))
    zp_ref[0] = lax.dot_general(
        proj, y_pred_ref[0], dn, preferred_element_type=jnp.float32)
    zr_ref[0] = lax.dot_general(
        proj, y_real_ref[0], dn, preferred_element_type=jnp.float32)


def _sc_sort_span(ref, cb):
    def p0(i, c):
        off = cb + i * 64
        for u in range(2):
            o = off + u * 32
            x = ref[pl.ds(o, 16)]
            ref[pl.ds(o, 16)] = plsc.sort_key_val(x, x)[0]
            y = ref[pl.ds(o + 16, 16)]
            ref[pl.ds(o + 16, 16)] = plsc.sort_key_val(y, y, descending=True)[0]
        return c

    lax.fori_loop(0, _VR // 4, p0, 0)

    k = 32
    while k <= _N:
        kv = k // 16
        j = k // 2
        while j >= 16:
            s = j // 16
            ls = s.bit_length() - 1

            def stage(i, c, s=s, ls=ls, kv=kv):
                for u in range(2):
                    iu = i * 2 + u
                    v_lo = ((iu >> ls) << (ls + 1)) + (iu & (s - 1))
                    dsc = jnp.where((v_lo & kv) != 0, s, 0)
                    oa = cb + v_lo * 16
                    ob = oa + s * 16
                    a = ref[pl.ds(oa, 16)]
                    b = ref[pl.ds(ob, 16)]
                    mn = jnp.minimum(a, b)
                    mx = jnp.maximum(a, b)
                    ref[pl.ds(oa + dsc * 16, 16)] = mn
                    ref[pl.ds(ob - dsc * 16, 16)] = mx
                return c

            lax.fori_loop(0, _VR // 4, stage, 0)
            j //= 2

        if k < _N:

            def clean(v, c, kv=kv):
                for u in range(4):
                    off = cb + (v * 4 + u) * 16
                    x = ref[pl.ds(off, 16)]
                    x = plsc.sort_key_val(x, x)[0]
                    flip = ((v * 4 + u) & kv) != 0
                    ref[pl.ds(off, 16)] = jnp.where(flip, jnp.flip(x), x)
                return c

        else:

            def clean(v, c):
                for u in range(4):
                    off = cb + (v * 4 + u) * 16
                    x = ref[pl.ds(off, 16)]
                    ref[pl.ds(off, 16)] = plsc.sort_key_val(x, x)[0]
                return c

        lax.fori_loop(0, _VR // 4, clean, 0)
        k *= 2


def _sc_swd(zp_hbm, zr_hbm, out_hbm, a_v, b_v, acc_v):
    wid = lax.axis_index("s") * _NC + lax.axis_index("c")
    base = wid * _CP * _N
    pltpu.sync_copy(zp_hbm.at[pl.ds(base, _CP * _N)], a_v)
    pltpu.sync_copy(zr_hbm.at[pl.ds(base, _CP * _N)], b_v)

    def col_body(col, acc):
        cb = col * _N
        _sc_sort_span(a_v, cb)
        _sc_sort_span(b_v, cb)

        def red(v, acc):
            off = cb + v * 16
            d = a_v[pl.ds(off, 16)] - b_v[pl.ds(off, 16)]
            return acc + d * d

        return lax.fori_loop(0, _VR, red, acc)

    acc = lax.fori_loop(0, _CP, col_body, jnp.zeros((16,), jnp.float32))
    acc_v[...] = acc
    pltpu.sync_copy(acc_v, out_hbm.at[pl.ds(wid * 16, 16)])


def _cmpex_rolled(x, j, up, keep_min):
    pu = jnp.concatenate([x[j:], x[:j]], axis=0)
    pd = jnp.concatenate([x[_N - j:], x[: _N - j]], axis=0)
    p = jnp.where(up, pu, pd)
    return jnp.where(keep_min, jnp.minimum(x, p), jnp.maximum(x, p))


def _cmpex_sliced(x, j, k):
    pieces = []
    for lo in range(0, _N, 2 * j):
        a = x[lo : lo + j]
        b = x[lo + j : lo + 2 * j]
        mn = jnp.minimum(a, b)
        mx = jnp.maximum(a, b)
        if lo & k:
            pieces += [mx, mn]
        else:
            pieces += [mn, mx]
    return jnp.concatenate(pieces, axis=0)


def _bitonic_sort_cols(x):
    iota = jax.lax.broadcasted_iota(jnp.int32, (_N, 1), 0)
    ups = {}
    bits = {}
    j = 1
    while j < _N:
        bits[j] = iota & j
        ups[j] = bits[j] == 0
        j *= 2
    k = 2
    while k <= _N:
        j = k // 2
        while j >= 1:
            if _N // (2 * j) <= 16:
                x = _cmpex_sliced(x, j, k)
            else:
                if k < _N:
                    keep_min = ups[j] != (bits[k] != 0)
                else:
                    keep_min = ups[j]
                x = _cmpex_rolled(x, j, ups[j], keep_min)
            j //= 2
        k *= 2
    return x


def _swd_kernel(y_pred_ref, y_real_ref, proj_ref, out_ref):
    b = pl.program_id(0)
    proj = proj_ref[...]
    zp = jnp.dot(y_pred_ref[0], proj, preferred_element_type=jnp.float32)
    zr = jnp.dot(y_real_ref[0], proj, preferred_element_type=jnp.float32)
    z = jnp.concatenate([zp, zr], axis=1)
    z = _bitonic_sort_cols(z)
    d = z[:, :_P] - z[:, _P:]
    s = jnp.sum(d * d).reshape(1, 1)

    @pl.when(b == 0)
    def _():
        out_ref[...] = jnp.zeros((1, 1), jnp.float32)

    out_ref[...] += s


def kernel(y_pred, y_real, proj_mat):
    nb = _B - _SCB
    zp, zr = pl.pallas_call(
        _proj_kernel,
        grid=(_SCB,),
        in_specs=[
            pl.BlockSpec((1, _N, _F), lambda b: (b, 0, 0)),
            pl.BlockSpec((1, _N, _F), lambda b: (b, 0, 0)),
            pl.BlockSpec((_F, _P), lambda b: (0, 0)),
        ],
        out_specs=[
            pl.BlockSpec((1, _P, _N), lambda b: (b, 0, 0)),
            pl.BlockSpec((1, _P, _N), lambda b: (b, 0, 0)),
        ],
        out_shape=[
            jax.ShapeDtypeStruct((_SCB, _P, _N), jnp.float32),
            jax.ShapeDtypeStruct((_SCB, _P, _N), jnp.float32),
        ],
    )(y_pred[nb:], y_real[nb:], proj_mat)

    sc_swd = functools.partial(
        pl.kernel,
        mesh=plsc.VectorSubcoreMesh(core_axis_name="c", subcore_axis_name="s"),
        compiler_params=pltpu.CompilerParams(needs_layout_passes=False),
        cost_estimate=pl.CostEstimate(
            flops=2_000_000_000, transcendentals=0, bytes_accessed=4_200_000),
        out_type=jax.ShapeDtypeStruct((_NW * 16,), jnp.float32),
        scratch_types=[
            pltpu.VMEM((_CP * _N,), jnp.float32),
            pltpu.VMEM((_CP * _N,), jnp.float32),
            pltpu.VMEM((16,), jnp.float32),
        ],
    )(_sc_swd)
    partials = sc_swd(zp.reshape(-1), zr.reshape(-1))

    tc_sum = pl.pallas_call(
        _swd_kernel,
        grid=(nb,),
        in_specs=[
            pl.BlockSpec((1, _N, _F), lambda b: (b, 0, 0)),
            pl.BlockSpec((1, _N, _F), lambda b: (b, 0, 0)),
            pl.BlockSpec((_F, _P), lambda b: (0, 0)),
        ],
        out_specs=pl.BlockSpec((1, 1), lambda b: (0, 0)),
        out_shape=jax.ShapeDtypeStruct((1, 1), jnp.float32),
    )(y_pred[:nb], y_real[:nb], proj_mat)

    total = tc_sum.reshape(()) + jnp.sum(partials)
    return total / (_B * _N * _P)


# speedup vs baseline: 1.2122x; 1.2122x over previous
import jax
import jax.numpy as jnp
from jax.experimental import pallas as pl
from jax.experimental.pallas import tpu as pltpu

_B = 4
_N = 2048
_F = 1024
_P = 128


def _cmpex_rolled(x, j, up, keep_min):
    pu = jnp.concatenate([x[j:], x[:j]], axis=0)
    pd = jnp.concatenate([x[_N - j:], x[: _N - j]], axis=0)
    p = jnp.where(up, pu, pd)
    return jnp.where(keep_min, jnp.minimum(x, p), jnp.maximum(x, p))


def _cmpex_sliced(x, j, k):
    pieces = []
    for lo in range(0, _N, 2 * j):
        a = x[lo : lo + j]
        b = x[lo + j : lo + 2 * j]
        mn = jnp.minimum(a, b)
        mx = jnp.maximum(a, b)
        if lo & k:
            pieces += [mx, mn]
        else:
            pieces += [mn, mx]
    return jnp.concatenate(pieces, axis=0)


def _bitonic_sort_cols(x):
    iota = jax.lax.broadcasted_iota(jnp.int32, (_N, 1), 0)
    ups = {}
    bits = {}
    j = 1
    while j < _N:
        bits[j] = iota & j
        ups[j] = bits[j] == 0
        j *= 2
    k = 2
    while k <= _N:
        j = k // 2
        while j >= 1:
            if _N // (2 * j) <= 64:
                x = _cmpex_sliced(x, j, k)
            else:
                if k < _N:
                    keep_min = ups[j] != (bits[k] != 0)
                else:
                    keep_min = ups[j]
                x = _cmpex_rolled(x, j, ups[j], keep_min)
            j //= 2
        k *= 2
    return x


def _swd_kernel(y_pred_ref, y_real_ref, proj_ref, out_ref):
    b = pl.program_id(0)
    proj = proj_ref[...]
    zp = jnp.dot(y_pred_ref[0], proj, preferred_element_type=jnp.float32)
    zr = jnp.dot(y_real_ref[0], proj, preferred_element_type=jnp.float32)
    z = jnp.concatenate([zp, zr], axis=1)
    z = _bitonic_sort_cols(z)
    d = z[:, :_P] - z[:, _P:]
    s = jnp.sum(d * d).reshape(1, 1)

    @pl.when(b == 0)
    def _():
        out_ref[...] = jnp.zeros((1, 1), jnp.float32)

    out_ref[...] += s


def kernel(y_pred, y_real, proj_mat):
    out = pl.pallas_call(
        _swd_kernel,
        grid=(_B,),
        in_specs=[
            pl.BlockSpec((1, _N, _F), lambda b: (b, 0, 0)),
            pl.BlockSpec((1, _N, _F), lambda b: (b, 0, 0)),
            pl.BlockSpec((_F, _P), lambda b: (0, 0)),
        ],
        out_specs=pl.BlockSpec((1, 1), lambda b: (0, 0)),
        out_shape=jax.ShapeDtypeStruct((1, 1), jnp.float32),
    )(y_pred, y_real, proj_mat)
    return (out / (_B * _N * _P)).reshape(())


# speedup vs baseline: 1.2207x; 1.0070x over previous
import jax
import jax.numpy as jnp
from jax.experimental import pallas as pl
from jax.experimental.pallas import tpu as pltpu

_B = 4
_N = 2048
_F = 1024
_P = 128


def _cmpex_rolled(x, j, up, keep_min):
    pu = jnp.concatenate([x[j:], x[:j]], axis=0)
    pd = jnp.concatenate([x[_N - j:], x[: _N - j]], axis=0)
    p = jnp.where(up, pu, pd)
    return jnp.where(keep_min, jnp.minimum(x, p), jnp.maximum(x, p))


def _cmpex_sliced(x, j, k):
    pieces = []
    for lo in range(0, _N, 2 * j):
        a = x[lo : lo + j]
        b = x[lo + j : lo + 2 * j]
        mn = jnp.minimum(a, b)
        mx = jnp.maximum(a, b)
        if lo & k:
            pieces += [mx, mn]
        else:
            pieces += [mn, mx]
    return jnp.concatenate(pieces, axis=0)


def _bitonic_sort_cols(x):
    iota = jax.lax.broadcasted_iota(jnp.int32, (_N, 1), 0)
    ups = {}
    bits = {}
    j = 1
    while j < _N:
        bits[j] = iota & j
        ups[j] = bits[j] == 0
        j *= 2
    k = 2
    while k <= _N:
        j = k // 2
        while j >= 1:
            if _N // (2 * j) <= 128:
                x = _cmpex_sliced(x, j, k)
            else:
                if k < _N:
                    keep_min = ups[j] != (bits[k] != 0)
                else:
                    keep_min = ups[j]
                x = _cmpex_rolled(x, j, ups[j], keep_min)
            j //= 2
        k *= 2
    return x


def _swd_kernel(y_pred_ref, y_real_ref, proj_ref, out_ref):
    b = pl.program_id(0)
    proj = proj_ref[...]
    zp = jnp.dot(y_pred_ref[0], proj, preferred_element_type=jnp.float32)
    zr = jnp.dot(y_real_ref[0], proj, preferred_element_type=jnp.float32)
    z = jnp.concatenate([zp, zr], axis=1)
    z = _bitonic_sort_cols(z)
    d = z[:, :_P] - z[:, _P:]
    s = jnp.sum(d * d).reshape(1, 1)

    @pl.when(b == 0)
    def _():
        out_ref[...] = jnp.zeros((1, 1), jnp.float32)

    out_ref[...] += s


def kernel(y_pred, y_real, proj_mat):
    out = pl.pallas_call(
        _swd_kernel,
        grid=(_B,),
        in_specs=[
            pl.BlockSpec((1, _N, _F), lambda b: (b, 0, 0)),
            pl.BlockSpec((1, _N, _F), lambda b: (b, 0, 0)),
            pl.BlockSpec((_F, _P), lambda b: (0, 0)),
        ],
        out_specs=pl.BlockSpec((1, 1), lambda b: (0, 0)),
        out_shape=jax.ShapeDtypeStruct((1, 1), jnp.float32),
    )(y_pred, y_real, proj_mat)
    return (out / (_B * _N * _P)).reshape(())
